# baseline (device time: 100952 ns/iter reference)
import jax
import jax.numpy as jnp
from jax import lax
from jax.experimental import pallas as pl
from jax.experimental.pallas import tpu as pltpu

N_DEV = 32
BLK = 128
GROUP = 4
K_BLK = GROUP * BLK
N_STEPS = N_DEV // GROUP
N_COLS = 8192


def kernel(x, w_mat):
    k_dim, k_per = x.shape
    _, n = w_mat.shape

    def body(x_ref, w_ref, out_ref, xrows_ref, send_sems, recv_sems):
        me = lax.axis_index("i")
        j = pl.program_id(0)

        @pl.when(j == 0)
        def _():
            out_ref[:, :] = jnp.zeros_like(out_ref)
            xrows_ref[:, pl.ds(me * BLK, BLK)] = x_ref[pl.ds(me * BLK, BLK), :]

        issue_step = jnp.maximum(me // GROUP - 1, 0)
        for t in range(N_DEV):
            @pl.when((j == issue_step) & (t != me))
            def _():
                rdma = pltpu.make_async_remote_copy(
                    src_ref=x_ref.at[pl.ds(t * BLK, BLK), :],
                    dst_ref=xrows_ref.at[:, pl.ds(me * BLK, BLK)],
                    send_sem=send_sems.at[t],
                    recv_sem=recv_sems.at[me],
                    device_id=(t,),
                    device_id_type=pl.DeviceIdType.MESH,
                )
                rdma.start()

        for s in range(N_DEV):
            @pl.when((j == s // GROUP) & (s != me))
            def _():
                recv = pltpu.make_async_remote_copy(
                    src_ref=x_ref.at[pl.ds(s * BLK, BLK), :],
                    dst_ref=xrows_ref.at[:, pl.ds(s * BLK, BLK)],
                    send_sem=send_sems.at[s],
                    recv_sem=recv_sems.at[s],
                    device_id=(s,),
                    device_id_type=pl.DeviceIdType.MESH,
                )
                recv.wait_recv()

        out_ref[:, :] += jnp.dot(
            xrows_ref[:, pl.ds(j * K_BLK, K_BLK)],
            w_ref[:, :],
            preferred_element_type=jnp.float32,
        )

        @pl.when(j == N_STEPS - 1)
        def _():
            for t in range(N_DEV):
                @pl.when(t != me)
                def _():
                    send = pltpu.make_async_remote_copy(
                        src_ref=x_ref.at[pl.ds(t * BLK, BLK), :],
                        dst_ref=xrows_ref.at[:, pl.ds(me * BLK, BLK)],
                        send_sem=send_sems.at[t],
                        recv_sem=recv_sems.at[me],
                        device_id=(t,),
                        device_id_type=pl.DeviceIdType.MESH,
                    )
                    send.wait_send()
            y = out_ref[:, :]
            out_ref[:, :] = y * jax.nn.sigmoid(y)

    return pl.pallas_call(
        body,
        grid=(N_STEPS,),
        in_specs=[
            pl.BlockSpec((k_dim, k_per), lambda j: (0, 0)),
            pl.BlockSpec((K_BLK, n), lambda j: (j, 0)),
        ],
        out_specs=pl.BlockSpec((BLK, n), lambda j: (0, 0)),
        out_shape=jax.ShapeDtypeStruct((BLK, n), jnp.float32),
        scratch_shapes=[
            pltpu.VMEM((BLK, k_dim), jnp.float32),
            pltpu.SemaphoreType.DMA((N_DEV,)),
            pltpu.SemaphoreType.DMA((N_DEV,)),
        ],
        compiler_params=pltpu.CompilerParams(
            vmem_limit_bytes=56 * 1024 * 1024,
            dimension_semantics=("arbitrary",),
        ),
    )(x, w_mat)


# device time: 85873 ns/iter; 1.1756x vs baseline; 1.1756x over previous
import jax
import jax.numpy as jnp
from jax import lax
from jax.experimental import pallas as pl
from jax.experimental.pallas import tpu as pltpu

N_DEV = 32
BLK = 128
HALF_SRC = 16
K_HALF = HALF_SRC * BLK
N_BLK = 1024
N_COLS = 8192


def kernel(x, w_mat):
    k_dim, k_per = x.shape
    _, n = w_mat.shape
    n_steps = n // N_BLK

    def body(x_ref, w_ref, out_ref, xrows_ref, acc_ref, send_sems, recv_sems):
        me = lax.axis_index("i")
        p = pl.program_id(0)
        nb = pl.program_id(1)

        my_half = me // HALF_SRC

        @pl.when((p == 0) & (nb == 0))
        def _():
            xrows_ref[:, pl.ds(me * BLK, BLK)] = x_ref[pl.ds(me * BLK, BLK), :]

        issue_nb = jnp.where(my_half == 0, 0, 1)
        for t in range(N_DEV):
            @pl.when((p == 0) & (nb == issue_nb) & (t != me))
            def _():
                rdma = pltpu.make_async_remote_copy(
                    src_ref=x_ref.at[pl.ds(t * BLK, BLK), :],
                    dst_ref=xrows_ref.at[:, pl.ds(me * BLK, BLK)],
                    send_sem=send_sems.at[t],
                    recv_sem=recv_sems.at[me],
                    device_id=(t,),
                    device_id_type=pl.DeviceIdType.MESH,
                )
                rdma.start()

        for s in range(N_DEV):
            @pl.when((p == s // HALF_SRC) & (nb == 0) & (s != me))
            def _():
                recv = pltpu.make_async_remote_copy(
                    src_ref=x_ref.at[pl.ds(s * BLK, BLK), :],
                    dst_ref=xrows_ref.at[:, pl.ds(s * BLK, BLK)],
                    send_sem=send_sems.at[s],
                    recv_sem=recv_sems.at[s],
                    device_id=(s,),
                    device_id_type=pl.DeviceIdType.MESH,
                )
                recv.wait_recv()

        partial = jnp.dot(
            xrows_ref[:, pl.ds(p * K_HALF, K_HALF)],
            w_ref[:, :],
            preferred_element_type=jnp.float32,
        )

        @pl.when(p == 0)
        def _():
            acc_ref[:, pl.ds(nb * N_BLK, N_BLK)] = partial

        @pl.when(p == 1)
        def _():
            y = acc_ref[:, pl.ds(nb * N_BLK, N_BLK)] + partial
            out_ref[:, :] = y * jax.nn.sigmoid(y)

        @pl.when((p == 1) & (nb == n_steps - 1))
        def _():
            for t in range(N_DEV):
                @pl.when(t != me)
                def _():
                    send = pltpu.make_async_remote_copy(
                        src_ref=x_ref.at[pl.ds(t * BLK, BLK), :],
                        dst_ref=xrows_ref.at[:, pl.ds(me * BLK, BLK)],
                        send_sem=send_sems.at[t],
                        recv_sem=recv_sems.at[me],
                        device_id=(t,),
                        device_id_type=pl.DeviceIdType.MESH,
                    )
                    send.wait_send()

    return pl.pallas_call(
        body,
        grid=(2, n_steps),
        in_specs=[
            pl.BlockSpec((k_dim, k_per), lambda p, nb: (0, 0)),
            pl.BlockSpec((K_HALF, N_BLK), lambda p, nb: (p, nb)),
        ],
        out_specs=pl.BlockSpec((BLK, N_BLK), lambda p, nb: (0, nb)),
        out_shape=jax.ShapeDtypeStruct((BLK, n), jnp.float32),
        scratch_shapes=[
            pltpu.VMEM((BLK, k_dim), jnp.float32),
            pltpu.VMEM((BLK, N_COLS), jnp.float32),
            pltpu.SemaphoreType.DMA((N_DEV,)),
            pltpu.SemaphoreType.DMA((N_DEV,)),
        ],
        compiler_params=pltpu.CompilerParams(
            vmem_limit_bytes=56 * 1024 * 1024,
            dimension_semantics=("arbitrary", "arbitrary"),
        ),
    )(x, w_mat)
